# baseline (speedup 1.0000x reference)
import jax
import jax.numpy as jnp
from jax import lax
from jax.experimental import pallas as pl
from jax.experimental.pallas import tpu as pltpu
from jax.experimental.pallas import tpu_sc as plsc

VOCAB = 100000
EMB_D = 100
BATCH = 16384
SEQ_LEN = 100
N_OUT = 50
LPAD = 128

NC = 2
NS = 16
L = 16
NW = NC * NS

ROWS_PER_BLK = 512
NSTRIPE = 4
ROWS_PER_STEP = ROWS_PER_BLK * NSTRIPE
NSTEP = -(-VOCAB // ROWS_PER_STEP)
VPAD = NSTEP * ROWS_PER_STEP


def _rowmean_body(*refs):
    emb_refs, out_ref = refs[:NSTRIPE], refs[NSTRIPE]
    ones_row = jnp.full((1, EMB_D), 1.0 / EMB_D, jnp.float32)
    pieces = []
    for s in range(NSTRIPE):
        pieces.append(lax.dot_general(
            ones_row, emb_refs[s][...],
            (((1,), (1,)), ((), ())),
            preferred_element_type=jnp.float32))
    out_ref[...] = jnp.concatenate(pieces, axis=1).reshape(ROWS_PER_STEP)


def _rowmean(emb_table):
    return pl.pallas_call(
        _rowmean_body,
        grid=(NSTEP,),
        in_specs=[
            pl.BlockSpec((ROWS_PER_BLK, EMB_D),
                         lambda i, s=s: (i * NSTRIPE + s, 0))
            for s in range(NSTRIPE)
        ],
        out_specs=pl.BlockSpec((ROWS_PER_STEP,), lambda i: (i,)),
        out_shape=jax.ShapeDtypeStruct((VPAD,), jnp.float32),
    )(*([emb_table] * NSTRIPE))


BX = 1024


def _pad_body(x_ref, o_ref):
    blk = x_ref[...]
    o_ref[...] = jnp.concatenate(
        [blk, jnp.zeros((BX, LPAD - SEQ_LEN), jnp.int32)], axis=1)


def _pad_x(x):
    return pl.pallas_call(
        _pad_body,
        grid=(BATCH // BX,),
        in_specs=[pl.BlockSpec((BX, SEQ_LEN), lambda i: (i, 0))],
        out_specs=pl.BlockSpec((BX, LPAD), lambda i: (i, 0)),
        out_shape=jax.ShapeDtypeStruct((BATCH, LPAD), jnp.int32),
    )(x)


ROWS_PER_W = BATCH // NW
CROWS = 64
NCHUNK = ROWS_PER_W // CROWS
ROW_OFFS = (0, 16, 32, 48, 64, 80, 84)


def _gather_body(tab_hbm, x_hbm, out_hbm, tab_v, idx_v, out_v):
    wid = lax.axis_index("s") * NC + lax.axis_index("c")
    base = wid * ROWS_PER_W
    pltpu.sync_copy(tab_hbm, tab_v)
    head = tab_v[pl.ds(0, L)]
    tab_v[pl.ds(0, L)] = jnp.where(
        lax.broadcasted_iota(jnp.int32, (L,), 0) == 1, 0.0, head)

    def chunk_body(c, _):
        r0 = base + c * CROWS
        pltpu.sync_copy(x_hbm.at[pl.ds(r0, CROWS), :], idx_v)

        def row_body(r, _):
            for off in ROW_OFFS:
                iv = idx_v[r, pl.ds(off, L)]
                out_v[r, pl.ds(off, L)] = plsc.load_gather(tab_v, [iv])
            return 0

        lax.fori_loop(0, CROWS, row_body, 0)
        pltpu.sync_copy(out_v, out_hbm.at[pl.ds(r0, CROWS), :])
        return 0

    lax.fori_loop(0, NCHUNK, chunk_body, 0)


_gather = pl.kernel(
    _gather_body,
    out_type=jax.ShapeDtypeStruct((BATCH, LPAD), jnp.float32),
    mesh=plsc.VectorSubcoreMesh(core_axis_name="c", subcore_axis_name="s"),
    compiler_params=pltpu.CompilerParams(needs_layout_passes=False,
                                         use_tc_tiling_on_sc=True),
    scratch_types=[
        pltpu.VMEM((VPAD,), jnp.float32),
        pltpu.VMEM((CROWS, LPAD), jnp.int32),
        pltpu.VMEM((CROWS, LPAD), jnp.float32),
    ],
)


BM = 1024


def _mm_body(m_ref, w_ref, o_ref):
    o_ref[...] = jnp.dot(m_ref[...][:, :SEQ_LEN], w_ref[...],
                         preferred_element_type=jnp.float32)


def _matmul(m2, w_t):
    return pl.pallas_call(
        _mm_body,
        grid=(BATCH // BM,),
        in_specs=[pl.BlockSpec((BM, LPAD), lambda i: (i, 0)),
                  pl.BlockSpec((SEQ_LEN, N_OUT), lambda i: (0, 0))],
        out_specs=pl.BlockSpec((BM, N_OUT), lambda i: (i, 0)),
        out_shape=jax.ShapeDtypeStruct((BATCH, N_OUT), jnp.float32),
    )(m2, w_t)


def kernel(x, x_len, mask, emb_table, W_final):
    del x_len, mask
    rowmean = _rowmean(emb_table)
    m = _gather(rowmean, _pad_x(x))
    return _matmul(m, W_final.T)


# speedup vs baseline: 1.8025x; 1.8025x over previous
import jax
import jax.numpy as jnp
from jax import lax
from jax.experimental import pallas as pl
from jax.experimental.pallas import tpu as pltpu
from jax.experimental.pallas import tpu_sc as plsc

VOCAB = 100000
EMB_D = 100
BATCH = 16384
SEQ_LEN = 100
N_OUT = 50
LPAD = 128

NC = 2
NS = 16
L = 16
NW = NC * NS

COLS_PER_BLK = 512
NSTRIPE = 4
COLS_PER_STEP = COLS_PER_BLK * NSTRIPE
NSTEP = -(-VOCAB // COLS_PER_STEP)
VPAD = NSTEP * COLS_PER_STEP


def _rowmean_body(*refs):
    emb_refs, out_ref = refs[:NSTRIPE], refs[NSTRIPE]
    ones_row = jnp.full((1, EMB_D), 1.0 / EMB_D, jnp.float32)
    for s in range(NSTRIPE):
        v = jnp.dot(ones_row, emb_refs[s][...],
                    preferred_element_type=jnp.float32)
        out_ref[pl.ds(s * COLS_PER_BLK, COLS_PER_BLK)] = (
            v.reshape(COLS_PER_BLK))


def _rowmean(emb_t):
    return pl.pallas_call(
        _rowmean_body,
        grid=(NSTEP,),
        in_specs=[
            pl.BlockSpec((EMB_D, COLS_PER_BLK),
                         lambda i, s=s: (0, i * NSTRIPE + s))
            for s in range(NSTRIPE)
        ],
        out_specs=pl.BlockSpec((COLS_PER_STEP,), lambda i: (i,)),
        out_shape=jax.ShapeDtypeStruct((VPAD,), jnp.float32),
    )(*([emb_t] * NSTRIPE))


ROWS_PER_W = BATCH // NW
CROWS = 32
NCHUNK = ROWS_PER_W // CROWS
ROW_OFFS = (0, 16, 32, 48, 64, 80, 84)


def _gather_body(tab_hbm, x_hbm, out_hbm, tab_v, idx_v, out_v,
                 sem_t, sem_i0, sem_i1, sem_o0, sem_o1):
    wid = lax.axis_index("s") * NC + lax.axis_index("c")
    base = wid * ROWS_PER_W
    sem_i = (sem_i0, sem_i1)
    sem_o = (sem_o0, sem_o1)

    def idx_copy(c):
        r0 = base + c * CROWS
        return pltpu.make_async_copy(
            x_hbm.at[pl.ds(r0, CROWS), :], idx_v.at[c % 2], sem_i[c % 2])

    def out_copy(c):
        r0 = base + c * CROWS
        return pltpu.make_async_copy(
            out_v.at[c % 2], out_hbm.at[pl.ds(r0, CROWS), :], sem_o[c % 2])

    tab_cp = pltpu.make_async_copy(tab_hbm, tab_v, sem_t)
    tab_cp.start()
    idx_copy(0).start()
    idx_copy(1).start()
    tab_cp.wait()
    head = tab_v[pl.ds(0, L)]
    tab_v[pl.ds(0, L)] = jnp.where(
        lax.broadcasted_iota(jnp.int32, (L,), 0) == 1, 0.0, head)

    for c in range(NCHUNK):
        slot = c % 2
        idx_copy(c).wait()
        if c >= 2:
            out_copy(c - 2).wait()

        def row_body(r, _):
            for off in ROW_OFFS:
                iv = idx_v[slot, r, pl.ds(off, L)]
                out_v[slot, r, pl.ds(off, L)] = plsc.load_gather(
                    tab_v, [iv])
            return 0

        lax.fori_loop(0, CROWS, row_body, 0)
        out_copy(c).start()
        if c + 2 < NCHUNK:
            idx_copy(c + 2).start()
    out_copy(NCHUNK - 2).wait()
    out_copy(NCHUNK - 1).wait()


_gather = pl.kernel(
    _gather_body,
    out_type=jax.ShapeDtypeStruct((BATCH, LPAD), jnp.float32),
    mesh=plsc.VectorSubcoreMesh(core_axis_name="c", subcore_axis_name="s"),
    compiler_params=pltpu.CompilerParams(needs_layout_passes=False),
    scratch_types=[
        pltpu.VMEM((VPAD,), jnp.float32),
        pltpu.VMEM((2, CROWS, SEQ_LEN), jnp.int32),
        pltpu.VMEM((2, CROWS, LPAD), jnp.float32),
        pltpu.SemaphoreType.DMA,
        pltpu.SemaphoreType.DMA,
        pltpu.SemaphoreType.DMA,
        pltpu.SemaphoreType.DMA,
        pltpu.SemaphoreType.DMA,
    ],
)


BM = 2048


def _mm_body(w_ref, m_ref, o_ref):
    o_ref[...] = lax.dot_general(
        w_ref[...], m_ref[...][:, :SEQ_LEN],
        (((1,), (1,)), ((), ())),
        preferred_element_type=jnp.float32)


def _matmul_t(w, m2):
    return pl.pallas_call(
        _mm_body,
        grid=(BATCH // BM,),
        in_specs=[pl.BlockSpec((N_OUT, SEQ_LEN), lambda i: (0, 0)),
                  pl.BlockSpec((BM, LPAD), lambda i: (i, 0))],
        out_specs=pl.BlockSpec((N_OUT, BM), lambda i: (0, i)),
        out_shape=jax.ShapeDtypeStruct((N_OUT, BATCH), jnp.float32),
    )(w, m2)


def kernel(x, x_len, mask, emb_table, W_final):
    del x_len, mask
    rowmean = _rowmean(emb_table.T)
    m = _gather(rowmean, x)
    return _matmul_t(W_final, m).T
